# baseline (speedup 1.0000x reference)
import functools

import jax
import jax.numpy as jnp
from jax import lax
from jax.experimental import pallas as pl
from jax.experimental.pallas import tpu as pltpu
from jax.experimental.pallas import tpu_sc as plsc

N = 10000
NP = 10240
E = 320000
IN_DIM = 128
HID = 16
HEADS = 8
OUT_DIM = 64

NUM_WORKERS = 32
EDGES_PER_WORKER = E // NUM_WORKERS
CHUNK = 40
NSTEPS = EDGES_PER_WORKER // CHUNK
ROWS_PER_TILE = NP // 16

_f32 = jnp.float32


def _prep1_body(x_ref, w_ref, ps_ref, pd_ref, h_ref, s_ref, d_ref, c_ref):
    h = jnp.dot(x_ref[...], w_ref[...], preferred_element_type=_f32)
    h_ref[...] = h
    s = jnp.dot(h, ps_ref[...], preferred_element_type=_f32)
    d = jnp.dot(h, pd_ref[...], preferred_element_type=_f32)
    s_ref[...] = s
    d_ref[...] = d
    c = jnp.max(s, axis=0, keepdims=True) + jnp.max(d, axis=0, keepdims=True)
    c_ref[...] = jnp.maximum(c, 0.0)


def _mid_body(p_ref, en_ref, ed_ref, b1_ref, w2_ref, ps_ref, pd_ref,
              h_ref, s_ref, d_ref, c_ref):
    p = p_ref[...]
    num = jnp.dot(p, en_ref[...], preferred_element_type=_f32)
    den = jnp.dot(p, ed_ref[...], preferred_element_type=_f32)
    h1 = jax.nn.relu(num / (den + 1e-16) + b1_ref[...])
    h2 = jnp.dot(h1, w2_ref[...], preferred_element_type=_f32)
    h_ref[...] = h2
    s = jnp.dot(h2, ps_ref[...], preferred_element_type=_f32)
    d = jnp.dot(h2, pd_ref[...], preferred_element_type=_f32)
    s_ref[...] = s
    d_ref[...] = d
    c = jnp.max(s, axis=0, keepdims=True) + jnp.max(d, axis=0, keepdims=True)
    c_ref[...] = jnp.maximum(c, 0.0)


def _final_body(q_ref, en_ref, ed_ref, b2_ref, o_ref):
    q = q_ref[...][:N]
    num = jnp.dot(q, en_ref[...], preferred_element_type=_f32)
    den = jnp.dot(q, ed_ref[...], preferred_element_type=_f32)
    o = num / (den + 1e-16) + b2_ref[...]
    m = jnp.max(o, axis=1, keepdims=True)
    e = jnp.exp(o - m)
    o_ref[...] = (o - m) - jnp.log(jnp.sum(e, axis=1, keepdims=True))


def _prep1(x, w1, ps1, pd1):
    return pl.pallas_call(
        _prep1_body,
        out_shape=[
            jax.ShapeDtypeStruct((N, HEADS * HID), _f32),
            jax.ShapeDtypeStruct((N, 16), _f32),
            jax.ShapeDtypeStruct((N, 16), _f32),
            jax.ShapeDtypeStruct((1, 16), _f32),
        ],
    )(x, w1, ps1, pd1)


def _mid(p, en1, ed1, b1, w2, ps2, pd2):
    return pl.pallas_call(
        _mid_body,
        out_shape=[
            jax.ShapeDtypeStruct((NP, OUT_DIM), _f32),
            jax.ShapeDtypeStruct((NP, 16), _f32),
            jax.ShapeDtypeStruct((NP, 16), _f32),
            jax.ShapeDtypeStruct((1, 16), _f32),
        ],
    )(p, en1, ed1, b1, w2, ps2, pd2)


def _final(q, en2, ed2, b2):
    return pl.pallas_call(
        _final_body,
        out_shape=jax.ShapeDtypeStruct((N, OUT_DIM), _f32),
    )(q, en2, ed2, b2)


SCH = 4000
NCH = E // SCH
GB = 128
RANGE = NP // NUM_WORKERS


def _make_edge_kernel(dw, chunk_lanes):
    rw = dw + 16
    mesh = plsc.VectorSubcoreMesh(core_axis_name="c", subcore_axis_name="s")

    @functools.partial(
        pl.kernel,
        mesh=mesh,
        compiler_params=pltpu.CompilerParams(use_tc_tiling_on_sc=False,
                                             needs_layout_passes=False),
        out_type=jax.ShapeDtypeStruct((NP, rw), _f32),
        scratch_types=[
            pltpu.VMEM((RANGE, rw), _f32),
            pltpu.VMEM((SCH,), jnp.int32),
            pltpu.VMEM((SCH,), jnp.int32),
            pltpu.VMEM((SCH,), jnp.int32),
            pltpu.VMEM((SCH,), jnp.int32),
            pltpu.VMEM((GB, dw), _f32),
            pltpu.VMEM((GB, 16), _f32),
            pltpu.VMEM((GB, 16), _f32),
            pltpu.VMEM((16,), _f32),
            pltpu.SemaphoreType.DMA,
        ],
    )
    def edge_kernel(h_hbm, s_hbm, d_hbm, c_hbm, src_hbm, dst_hbm, out_hbm,
                    acc, svbuf, dvbuf, list_s, list_d, hbuf, sbuf, dbuf,
                    cbuf, sem):
        cid = lax.axis_index("c")
        sid = lax.axis_index("s")
        wid = sid * 2 + cid
        lo = wid * RANGE

        pltpu.sync_copy(c_hbm, cbuf)
        cvec = cbuf[...]
        zv = jnp.zeros((16,), _f32)
        zi = jnp.zeros((16,), jnp.int32)

        def zero_acc(r, carry):
            for j in range(rw // 16):
                acc[r, pl.ds(j * 16, 16)] = zv
            return carry

        lax.fori_loop(0, RANGE, zero_acc, 0)

        def zero_lists(g, carry):
            list_s[pl.ds(g * 16, 16)] = zi
            list_d[pl.ds(g * 16, 16)] = zi
            return carry

        lax.fori_loop(0, SCH // 16, zero_lists, 0)

        lanes = jnp.arange(16, dtype=jnp.int32)

        def chunk_body(ci, carry):
            pltpu.sync_copy(src_hbm.at[pl.ds(ci * SCH, SCH)], svbuf)
            pltpu.sync_copy(dst_hbm.at[pl.ds(ci * SCH, SCH)], dvbuf)

            def scan_group(g, ptr):
                sv = svbuf[pl.ds(g * 16, 16)]
                dv = dvbuf[pl.ds(g * 16, 16)]
                m = (dv >= lo) & (dv < lo + RANGE)
                mi = jnp.where(m, jnp.int32(1), jnp.int32(0))
                idxv = ptr + plsc.cumsum(mi) - 1
                plsc.store_scatter(list_s, [idxv], sv, mask=m)
                plsc.store_scatter(list_d, [idxv], dv, mask=m)
                return ptr + plsc.all_reduce_population_count(m)

            ptr = lax.fori_loop(0, SCH // 16, scan_group, zi)
            t_cnt = ptr[0]
            nb = (t_cnt + GB - 1) // GB

            def batch_body(j, carry2):
                off = j * GB
                pltpu.async_copy(h_hbm.at[list_s.at[pl.ds(off, GB)]],
                                 hbuf, sem)
                pltpu.async_copy(s_hbm.at[list_s.at[pl.ds(off, GB)]],
                                 sbuf, sem)
                pltpu.async_copy(d_hbm.at[list_d.at[pl.ds(off, GB)]],
                                 dbuf, sem)
                pltpu.make_async_copy(h_hbm.at[list_s.at[pl.ds(off, GB)]],
                                      hbuf, sem).wait()
                pltpu.make_async_copy(s_hbm.at[list_s.at[pl.ds(off, GB)]],
                                      sbuf, sem).wait()
                pltpu.make_async_copy(d_hbm.at[list_d.at[pl.ds(off, GB)]],
                                      dbuf, sem).wait()

                def group_body(gg, carry3):
                    row0 = gg * 16
                    dvec = list_d[pl.ds(off + row0, 16)]
                    rvec = jnp.clip(dvec - lo, 0, RANGE - 1)
                    validv = jnp.where((lanes + (off + row0)) < t_cnt,
                                       jnp.float32(1.0), jnp.float32(0.0))
                    for l in range(16):
                        r_l = rvec[l]
                        alpha = sbuf[row0 + l] + dbuf[row0 + l]
                        alpha = jnp.where(alpha >= 0.0, alpha, alpha * 0.2)
                        ex = jnp.exp(alpha - cvec)
                        exm = validv[l] * ex
                        for j2, lane in enumerate(chunk_lanes):
                            a = acc[r_l, pl.ds(j2 * 16, 16)]
                            acc[r_l, pl.ds(j2 * 16, 16)] = \
                                a + exm[lane] * hbuf[row0 + l,
                                                     pl.ds(j2 * 16, 16)]
                        aden = acc[r_l, pl.ds(dw, 16)]
                        acc[r_l, pl.ds(dw, 16)] = aden + exm
                    return carry3

                lax.fori_loop(0, GB // 16, group_body, 0)
                return carry2

            lax.fori_loop(0, nb, batch_body, 0)
            return carry

        lax.fori_loop(0, NCH, chunk_body, 0)
        pltpu.sync_copy(acc, out_hbm.at[pl.ds(lo, RANGE)])

    return edge_kernel


_edge1 = _make_edge_kernel(HEADS * HID, tuple(range(HEADS)))
_edge2 = _make_edge_kernel(OUT_DIM, (0,) * (OUT_DIM // 16))


def kernel(x, edge_index, W1, att_src1, att_dst1, b1, W2, att_src2, att_dst2,
           b2):
    src = edge_index[0].astype(jnp.int32)
    dst = edge_index[1].astype(jnp.int32)

    head_of = jnp.arange(HEADS * HID) // HID
    oh16 = jax.nn.one_hot(head_of, 16, dtype=_f32)
    ps1 = att_src1.reshape(HEADS * HID, 1) * oh16
    pd1 = att_dst1.reshape(HEADS * HID, 1) * oh16
    en1 = jnp.concatenate(
        [jnp.eye(HEADS * HID, dtype=_f32), jnp.zeros((16, HEADS * HID), _f32)], 0)
    ed1 = jnp.concatenate(
        [jnp.zeros((HEADS * HID, HEADS * HID), _f32),
         jax.nn.one_hot(head_of, HEADS, dtype=_f32).T,
         jnp.zeros((16 - HEADS, HEADS * HID), _f32)], 0)
    ps2 = jnp.pad(att_src2.T, ((0, 0), (0, 15)))
    pd2 = jnp.pad(att_dst2.T, ((0, 0), (0, 15)))
    en2 = jnp.concatenate(
        [jnp.eye(OUT_DIM, dtype=_f32), jnp.zeros((16, OUT_DIM), _f32)], 0)
    ed2 = jnp.concatenate(
        [jnp.zeros((OUT_DIM, OUT_DIM), _f32), jnp.ones((1, OUT_DIM), _f32),
         jnp.zeros((15, OUT_DIM), _f32)], 0)

    h1, s1, d1, c1 = _prep1(x, W1, ps1, pd1)
    p = _edge1(h1, s1, d1, c1.reshape(16), src, dst)

    h2, s2, d2, c2 = _mid(p, en1, ed1, b1, W2, ps2, pd2)
    q = _edge2(h2, s2, d2, c2.reshape(16), src, dst)

    return _final(q, en2, ed2, b2)


# speedup vs baseline: 5.6288x; 5.6288x over previous
import functools

import jax
import jax.numpy as jnp
from jax import lax
from jax.experimental import pallas as pl
from jax.experimental.pallas import tpu as pltpu
from jax.experimental.pallas import tpu_sc as plsc

N = 10000
NP = 10240
E = 320000
IN_DIM = 128
HID = 16
HEADS = 8
OUT_DIM = 64

NUM_WORKERS = 32
EDGES_PER_WORKER = E // NUM_WORKERS
CHUNK = 40
NSTEPS = EDGES_PER_WORKER // CHUNK
ROWS_PER_TILE = NP // 16

_f32 = jnp.float32


def _prep1_body(x_ref, w_ref, ps_ref, pd_ref, h_ref, s_ref, d_ref, c_ref):
    h = jnp.dot(x_ref[...], w_ref[...], preferred_element_type=_f32)
    h_ref[...] = h
    s = jnp.dot(h, ps_ref[...], preferred_element_type=_f32)
    d = jnp.dot(h, pd_ref[...], preferred_element_type=_f32)
    s_ref[...] = s
    d_ref[...] = d
    c = jnp.max(s, axis=0, keepdims=True) + jnp.max(d, axis=0, keepdims=True)
    c_ref[...] = jnp.maximum(c, 0.0)


def _mid_body(p0_ref, p1_ref, en_ref, ed_ref, b1_ref, w2_ref, ps_ref, pd_ref,
              h_ref, s_ref, d_ref, c_ref):
    p = p0_ref[...] + p1_ref[...]
    num = jnp.dot(p, en_ref[...], preferred_element_type=_f32)
    den = jnp.dot(p, ed_ref[...], preferred_element_type=_f32)
    h1 = jax.nn.relu(num / (den + 1e-16) + b1_ref[...])
    h2 = jnp.dot(h1, w2_ref[...], preferred_element_type=_f32)
    h_ref[...] = h2
    s = jnp.dot(h2, ps_ref[...], preferred_element_type=_f32)
    d = jnp.dot(h2, pd_ref[...], preferred_element_type=_f32)
    s_ref[...] = s
    d_ref[...] = d
    c = jnp.max(s, axis=0, keepdims=True) + jnp.max(d, axis=0, keepdims=True)
    c_ref[...] = jnp.maximum(c, 0.0)


def _final_body(q0_ref, q1_ref, en_ref, ed_ref, b2_ref, o_ref):
    q = q0_ref[...][:N] + q1_ref[...][:N]
    num = jnp.dot(q, en_ref[...], preferred_element_type=_f32)
    den = jnp.dot(q, ed_ref[...], preferred_element_type=_f32)
    o = num / (den + 1e-16) + b2_ref[...]
    m = jnp.max(o, axis=1, keepdims=True)
    e = jnp.exp(o - m)
    o_ref[...] = (o - m) - jnp.log(jnp.sum(e, axis=1, keepdims=True))


def _prep1(x, w1, ps1, pd1):
    return pl.pallas_call(
        _prep1_body,
        out_shape=[
            jax.ShapeDtypeStruct((N, HEADS * HID), _f32),
            jax.ShapeDtypeStruct((N, 16), _f32),
            jax.ShapeDtypeStruct((N, 16), _f32),
            jax.ShapeDtypeStruct((1, 16), _f32),
        ],
    )(x, w1, ps1, pd1)


def _mid(p0, p1, en1, ed1, b1, w2, ps2, pd2):
    return pl.pallas_call(
        _mid_body,
        out_shape=[
            jax.ShapeDtypeStruct((NP, OUT_DIM), _f32),
            jax.ShapeDtypeStruct((NP, 16), _f32),
            jax.ShapeDtypeStruct((NP, 16), _f32),
            jax.ShapeDtypeStruct((1, 16), _f32),
        ],
    )(p0, p1, en1, ed1, b1, w2, ps2, pd2)


def _final(q0, q1, en2, ed2, b2):
    return pl.pallas_call(
        _final_body,
        out_shape=jax.ShapeDtypeStruct((N, OUT_DIM), _f32),
    )(q0, q1, en2, ed2, b2)


def _make_edge_kernel(dw, chunk_lanes):
    rw = dw + 16
    mesh = plsc.VectorSubcoreMesh(core_axis_name="c", subcore_axis_name="s")

    @functools.partial(
        pl.kernel,
        mesh=mesh,
        compiler_params=pltpu.CompilerParams(use_tc_tiling_on_sc=False),
        out_type=[
            jax.ShapeDtypeStruct((NP, rw), _f32),
            jax.ShapeDtypeStruct((NP, rw), _f32),
        ],
        scratch_types=[
            pltpu.VMEM_SHARED((NP, rw), _f32),
            pltpu.VMEM((CHUNK,), jnp.int32),
            pltpu.VMEM((CHUNK,), jnp.int32),
            pltpu.VMEM((CHUNK,), jnp.int32),
            pltpu.VMEM((CHUNK,), jnp.int32),
            pltpu.VMEM((CHUNK, dw), _f32),
            pltpu.VMEM((CHUNK, dw), _f32),
            pltpu.VMEM((CHUNK, 16), _f32),
            pltpu.VMEM((CHUNK, 16), _f32),
            pltpu.VMEM((CHUNK, 16), _f32),
            pltpu.VMEM((CHUNK, 16), _f32),
            pltpu.VMEM((CHUNK, rw), _f32),
            pltpu.VMEM((CHUNK, rw), _f32),
            pltpu.VMEM((16,), _f32),
            pltpu.SemaphoreType.DMA,
            pltpu.SemaphoreType.DMA,
            pltpu.SemaphoreType.DMA,
            pltpu.SemaphoreType.DMA,
        ],
    )
    def edge_kernel(h_hbm, s_hbm, d_hbm, c_hbm, src_hbm, dst_hbm, zero_hbm,
                    out0, out1, acc, isrc0, isrc1, idst0, idst1, hbuf0, hbuf1,
                    sbuf0, sbuf1, dbuf0, dbuf1, mbuf0, mbuf1, cbuf,
                    sg0, sg1, ss0, ss1):
        cid = lax.axis_index("c")
        sid = lax.axis_index("s")
        wid = sid * 2 + cid
        r0 = sid * ROWS_PER_TILE

        pltpu.sync_copy(zero_hbm.at[pl.ds(r0, ROWS_PER_TILE)],
                        acc.at[pl.ds(r0, ROWS_PER_TILE)])
        pltpu.sync_copy(c_hbm, cbuf)
        plsc.subcore_barrier()

        ebase = wid * EDGES_PER_WORKER
        cvec = cbuf[...]
        isrc = (isrc0, isrc1)
        idst = (idst0, idst1)
        hb = (hbuf0, hbuf1)
        sb = (sbuf0, sbuf1)
        db = (dbuf0, dbuf1)
        mb = (mbuf0, mbuf1)
        sg = (sg0, sg1)
        ss = (ss0, ss1)

        def start(gi, b):
            base = ebase + gi * CHUNK
            pltpu.sync_copy(src_hbm.at[pl.ds(base, CHUNK)], isrc[b])
            pltpu.sync_copy(dst_hbm.at[pl.ds(base, CHUNK)], idst[b])
            pltpu.async_copy(h_hbm.at[isrc[b]], hb[b], sg[b])
            pltpu.async_copy(s_hbm.at[isrc[b]], sb[b], sg[b])
            pltpu.async_copy(d_hbm.at[idst[b]], db[b], sg[b])

        def wait_gathers(b):
            pltpu.make_async_copy(h_hbm.at[isrc[b]], hb[b], sg[b]).wait()
            pltpu.make_async_copy(s_hbm.at[isrc[b]], sb[b], sg[b]).wait()
            pltpu.make_async_copy(d_hbm.at[idst[b]], db[b], sg[b]).wait()

        def compute(b):
            hbuf, sbuf, dbuf, mbuf = hb[b], sb[b], db[b], mb[b]

            def edge_body(e, c2):
                alpha = sbuf[e] + dbuf[e]
                alpha = jnp.where(alpha >= 0.0, alpha, alpha * 0.2)
                ex = jnp.exp(alpha - cvec)
                mbuf[e, pl.ds(dw, 16)] = ex
                for j, lane in enumerate(chunk_lanes):
                    mbuf[e, pl.ds(j * 16, 16)] = \
                        ex[lane] * hbuf[e, pl.ds(j * 16, 16)]
                return c2

            lax.fori_loop(0, CHUNK, edge_body, 0, unroll=4)

        def scatter(b):
            pltpu.async_copy(mb[b], acc.at[idst[b]], ss[b], add=True)

        def wait_scatter(b):
            pltpu.make_async_copy(mb[b], acc.at[idst[b]], ss[b]).wait()

        start(0, 0)

        def outer(i, carry):
            for b in (0, 1):
                gi = 2 * i + b

                @pl.when(gi < NSTEPS - 1)
                def _():
                    start(gi + 1, 1 - b)

                wait_gathers(b)

                @pl.when(gi >= 2)
                def _():
                    wait_scatter(b)

                compute(b)
                scatter(b)
            return carry

        lax.fori_loop(0, NSTEPS // 2, outer, 0)
        wait_scatter(0)
        wait_scatter(1)
        plsc.subcore_barrier()

        @pl.when(cid == 0)
        def _():
            pltpu.sync_copy(acc.at[pl.ds(r0, ROWS_PER_TILE)],
                            out0.at[pl.ds(r0, ROWS_PER_TILE)])

        @pl.when(cid == 1)
        def _():
            pltpu.sync_copy(acc.at[pl.ds(r0, ROWS_PER_TILE)],
                            out1.at[pl.ds(r0, ROWS_PER_TILE)])

    return edge_kernel


_edge1 = _make_edge_kernel(HEADS * HID, tuple(range(HEADS)))
_edge2 = _make_edge_kernel(OUT_DIM, (0,) * (OUT_DIM // 16))


def kernel(x, edge_index, W1, att_src1, att_dst1, b1, W2, att_src2, att_dst2,
           b2):
    src = edge_index[0].astype(jnp.int32)
    dst = edge_index[1].astype(jnp.int32)

    head_of = jnp.arange(HEADS * HID) // HID
    oh16 = jax.nn.one_hot(head_of, 16, dtype=_f32)
    ps1 = att_src1.reshape(HEADS * HID, 1) * oh16
    pd1 = att_dst1.reshape(HEADS * HID, 1) * oh16
    en1 = jnp.concatenate(
        [jnp.eye(HEADS * HID, dtype=_f32), jnp.zeros((16, HEADS * HID), _f32)], 0)
    ed1 = jnp.concatenate(
        [jnp.zeros((HEADS * HID, HEADS * HID), _f32),
         jax.nn.one_hot(head_of, HEADS, dtype=_f32).T,
         jnp.zeros((16 - HEADS, HEADS * HID), _f32)], 0)
    ps2 = jnp.pad(att_src2.T, ((0, 0), (0, 15)))
    pd2 = jnp.pad(att_dst2.T, ((0, 0), (0, 15)))
    en2 = jnp.concatenate(
        [jnp.eye(OUT_DIM, dtype=_f32), jnp.zeros((16, OUT_DIM), _f32)], 0)
    ed2 = jnp.concatenate(
        [jnp.zeros((OUT_DIM, OUT_DIM), _f32), jnp.ones((1, OUT_DIM), _f32),
         jnp.zeros((15, OUT_DIM), _f32)], 0)

    h1, s1, d1, c1 = _prep1(x, W1, ps1, pd1)
    z1 = jnp.zeros((NP, HEADS * HID + 16), _f32)
    p0, p1 = _edge1(h1, s1, d1, c1.reshape(16), src, dst, z1)

    h2, s2, d2, c2 = _mid(p0, p1, en1, ed1, b1, W2, ps2, pd2)
    z2 = jnp.zeros((NP, OUT_DIM + 16), _f32)
    q0, q1 = _edge2(h2, s2, d2, c2.reshape(16), src, dst, z2)

    return _final(q0, q1, en2, ed2, b2)
